# baseline (speedup 1.0000x reference)
import jax
import jax.numpy as jnp
from jax import lax
from jax.experimental import pallas as pl
from jax.experimental.pallas import tpu as pltpu
from jax.experimental.pallas import tpu_sc as plsc

_L = 16


def _sc_body(upd_hbm, i1_hbm, i2_hbm, params_hbm, out_hbm,
             upd_v, i1_v, i2_v, buf_v):
    cid = lax.axis_index("c")
    sid = lax.axis_index("s")

    @pl.when(jnp.logical_and(cid == 0, sid == 0))
    def _():
        pltpu.sync_copy(params_hbm, buf_v)
        pltpu.sync_copy(upd_hbm, upd_v)
        pltpu.sync_copy(i1_hbm, i1_v)
        pltpu.sync_copy(i2_hbm, i2_v)
        lane = lax.iota(jnp.int32, _L)
        mask = lane < 2
        flat = i1_v[...] * 40 + (lane + 1) * 10 + i2_v[...]
        flat = jnp.where(mask, flat, 0)
        plsc.store_scatter(buf_v, [flat], upd_v[...], mask=mask)
        pltpu.sync_copy(buf_v, out_hbm)


def kernel(update, index1, index2, params):
    pad = jnp.zeros((_L - 2,), jnp.int32)
    upd16 = jnp.concatenate([update.astype(jnp.float32),
                             pad.astype(jnp.float32)])
    i116 = jnp.concatenate([index1.astype(jnp.int32), pad])
    i216 = jnp.concatenate([index2.astype(jnp.int32), pad])
    mesh = plsc.VectorSubcoreMesh(core_axis_name="c", subcore_axis_name="s")
    out = pl.kernel(
        _sc_body,
        mesh=mesh,
        compiler_params=pltpu.CompilerParams(needs_layout_passes=False),
        out_type=jax.ShapeDtypeStruct((160,), jnp.float32),
        scratch_types=[
            pltpu.VMEM((_L,), jnp.float32),
            pltpu.VMEM((_L,), jnp.int32),
            pltpu.VMEM((_L,), jnp.int32),
            pltpu.VMEM((160,), jnp.float32),
        ],
    )(upd16, i116, i216, params.reshape(160))
    return out.reshape(4, 4, 10)


# speedup vs baseline: 1.2035x; 1.2035x over previous
import jax
import jax.numpy as jnp
from jax import lax
from jax.experimental import pallas as pl
from jax.experimental.pallas import tpu as pltpu
from jax.experimental.pallas import tpu_sc as plsc

_L = 16


def _sc_body(upd_hbm, i1_hbm, i2_hbm, params_hbm, out_hbm,
             upd_v, i1_v, i2_v, buf_v, sem):
    sid = lax.axis_index("s")

    @pl.when(sid == 0)
    def _():
        cp_p = pltpu.make_async_copy(params_hbm, buf_v, sem)
        cp_u = pltpu.make_async_copy(upd_hbm, upd_v.at[pl.ds(0, 2)], sem)
        cp_1 = pltpu.make_async_copy(i1_hbm, i1_v.at[pl.ds(0, 2)], sem)
        cp_2 = pltpu.make_async_copy(i2_hbm, i2_v.at[pl.ds(0, 2)], sem)
        cp_p.start(); cp_u.start(); cp_1.start(); cp_2.start()
        cp_p.wait(); cp_u.wait(); cp_1.wait(); cp_2.wait()
        lane = lax.iota(jnp.int32, _L)
        mask = lane < 2
        flat = i1_v[...] * 40 + (lane + 1) * 10 + i2_v[...]
        flat = jnp.where(mask, flat, 0)
        plsc.store_scatter(buf_v, [flat], upd_v[...], mask=mask)
        pltpu.sync_copy(buf_v, out_hbm)


def kernel(update, index1, index2, params):
    mesh = plsc.VectorSubcoreMesh(core_axis_name="c", subcore_axis_name="s",
                                  num_cores=1)
    out = pl.kernel(
        _sc_body,
        mesh=mesh,
        compiler_params=pltpu.CompilerParams(needs_layout_passes=False),
        out_type=jax.ShapeDtypeStruct((160,), jnp.float32),
        scratch_types=[
            pltpu.VMEM((_L,), jnp.float32),
            pltpu.VMEM((_L,), jnp.int32),
            pltpu.VMEM((_L,), jnp.int32),
            pltpu.VMEM((160,), jnp.float32),
            pltpu.SemaphoreType.DMA,
        ],
    )(update.astype(jnp.float32), index1.astype(jnp.int32),
      index2.astype(jnp.int32), params.reshape(160))
    return out.reshape(4, 4, 10)
